# baseline (speedup 1.0000x reference)
import functools

import jax
import jax.numpy as jnp
from jax import lax
from jax.experimental import pallas as pl
from jax.experimental.pallas import tpu as pltpu
from jax.experimental.pallas import tpu_sc as plsc

D = 64
NE = 1000000
NPAIR = NE // 2
W1 = 256
NBLK_MAIN = NE // W1
TAIL0 = NBLK_MAIN * W1
NTAIL = NE - TAIL0
NB = 16384
NT = 20
BT = 128
NU = NT * 4


@jax.jit
def _embedding_lookup(ids_t, table_t, tail_t):
    info = plsc.get_sparse_core_info()
    num_cores, num_subcores = info.num_cores, info.num_subcores
    nw = num_cores * num_subcores
    n_k = NBLK_MAIN // nw + 2
    mesh = plsc.VectorSubcoreMesh(core_axis_name="c", subcore_axis_name="s")

    @functools.partial(
        pl.kernel,
        mesh=mesh,
        out_type=jax.ShapeDtypeStruct((NT, 8, NB // BT, 8, BT), jnp.float32),
        scratch_types=[
            pltpu.HBM((NPAIR, 2 * D), jnp.float32),
            pltpu.VMEM((D, W1), jnp.float32),
            pltpu.VMEM((D, W1), jnp.float32),
            pltpu.VMEM((W1 // 2, 2 * D), jnp.float32),
            pltpu.VMEM((W1 // 2, 2 * D), jnp.float32),
            pltpu.VMEM((NT, 4 * BT), jnp.int32),
            pltpu.VMEM((BT,), jnp.int32),
            pltpu.VMEM((BT,), jnp.int32),
            pltpu.VMEM((BT, 2 * D), jnp.float32),
            pltpu.VMEM((BT, 2 * D), jnp.float32),
            pltpu.VMEM((8, 8, BT), jnp.float32),
            pltpu.VMEM((8, 8, BT), jnp.float32),
            pltpu.SemaphoreType.DMA,
            pltpu.SemaphoreType.DMA,
            pltpu.SemaphoreType.DMA,
            pltpu.SemaphoreType.DMA,
            pltpu.SemaphoreType.DMA,
            pltpu.SemaphoreType.DMA,
            pltpu.SemaphoreType.DMA,
            pltpu.SemaphoreType.DMA,
            pltpu.SemaphoreType.REGULAR,
        ],
        compiler_params=pltpu.CompilerParams(
            use_tc_tiling_on_sc=True, needs_layout_passes=False),
    )
    def emb(ids_hbm, table_hbm, tail_hbm, out_hbm, r_hbm,
            slab0, slab1, rowbuf0, rowbuf1, idxall, idxp0, idxp1,
            rows0, rows1, oslab0, oslab1,
            sr0, sr1, sw0, sw1, g0, g1, o0, o1, csem):
        cid = lax.axis_index("c")
        sid = lax.axis_index("s")
        wid = sid * num_cores + cid

        slabs = (slab0, slab1)
        rowbufs = (rowbuf0, rowbuf1)
        srs = (sr0, sr1)
        sws = (sw0, sw1)
        idxps = (idxp0, idxp1)
        rowss = (rows0, rows1)
        oslabs = (oslab0, oslab1)
        gs = (g0, g1)
        os_ = (o0, o1)

        iota = lax.iota(jnp.int32, 16)
        half = iota >> 1
        parity64 = (iota & 1) << 6
        def rot(d):
            return (iota + d) & 15

        def blk_of(k):
            return wid + k * nw

        def fire_read(k, p):
            i0 = pl.multiple_of(blk_of(k) * W1, W1)
            pltpu.async_copy(table_hbm.at[:, pl.ds(i0, W1)], slabs[p], srs[p])

        def wait_read(p):
            pltpu.make_async_copy(
                table_hbm.at[:, pl.ds(0, W1)], slabs[p], srs[p]).wait()

        def fire_write(k, p):
            r0 = pl.multiple_of(blk_of(k) * (W1 // 2), W1 // 2)
            pltpu.async_copy(
                rowbufs[p], r_hbm.at[pl.ds(r0, W1 // 2), :], sws[p])

        def wait_write(p):
            pltpu.make_async_copy(
                rowbufs[p], r_hbm.at[pl.ds(0, W1 // 2), :], sws[p]).wait()

        def xpose_block(p):
            slab, rowbuf = slabs[p], rowbufs[p]

            def per_i0(i16, carry):
                i0 = i16 * 16
                ld_col = i0 + iota
                st_row = (i16 * 8) + half

                @plsc.parallel_loop(0, D, unroll=16)
                def _(m):
                    q = ((iota + m) & 15) + (m & 48)
                    v = plsc.load_gather(slab, [q, ld_col])
                    plsc.store_scatter(rowbuf, [st_row, parity64 + q], v)

                return carry

            lax.fori_loop(0, W1 // 16, per_i0, 0)

        @pl.when(blk_of(0) < NBLK_MAIN)
        def _():
            fire_read(0, 0)

        def p1_pair(i, carry):
            for p in range(2):
                k = i * 2 + p
                blk = blk_of(k)

                @pl.when(blk_of(k + 1) < NBLK_MAIN)
                def _():
                    fire_read(k + 1, 1 - p)

                @pl.when(blk < NBLK_MAIN)
                def _():
                    wait_read(p)

                    @pl.when(k >= 2)
                    def _():
                        wait_write(p)

                    xpose_block(p)
                    fire_write(k, p)

            return carry

        lax.fori_loop(0, n_k // 2, p1_pair, 0)
        wait_write(0)
        wait_write(1)

        @pl.when(wid == 0)
        def _tail():
            pltpu.sync_copy(tail_hbm, slab0.at[:, pl.ds(0, BT)])

            def tail_j(j, carry):
                colbase = parity64 + j
                for c0 in range(0, NTAIL, 16):
                    v = slab0[j, pl.ds(c0, 16)]
                    plsc.store_scatter(
                        rowbuf0, [(c0 >> 1) + half, colbase], v)
                return carry

            lax.fori_loop(0, D, tail_j, 0)
            pltpu.sync_copy(
                rowbuf0.at[pl.ds(0, NTAIL // 2), :],
                r_hbm.at[pl.ds(TAIL0 // 2, NTAIL // 2), :])

        plsc.subcore_barrier()
        pltpu.core_barrier(csem, core_axis_name="c")

        pltpu.sync_copy(ids_hbm.at[:, pl.ds(wid * (4 * BT), 4 * BT)], idxall)

        def prep_unit(u, p):
            t = u >> 2
            cbase = (u & 3) * BT

            def shift_blk(c, carry):
                idxps[p][pl.ds(c * 16, 16)] = (
                    idxall[t, pl.ds(cbase + c * 16, 16)] >> 1)
                return carry

            lax.fori_loop(0, BT // 16, shift_blk, 0)

        def fire_gather(p):
            pltpu.async_copy(r_hbm.at[idxps[p]], rowss[p], gs[p])

        def wait_gather(p):
            pltpu.make_async_copy(
                r_hbm.at[pl.ds(0, BT), :], rowss[p], gs[p]).wait()

        def fire_out(u, p):
            t = u >> 2
            tc = wid * 4 + (u & 3)
            pltpu.async_copy(oslabs[p], out_hbm.at[t, :, tc], os_[p])

        def wait_out(p):
            pltpu.make_async_copy(
                oslabs[p], out_hbm.at[0, :, 0], os_[p]).wait()

        def xpose_unit(u, p):
            t = u >> 2
            cbase = (u & 3) * BT
            rows, oslab = rowss[p], oslabs[p]

            def per_cb(ci, carry):
                cb0 = ci * 16
                hvec = (idxall[t, pl.ds(cbase + cb0, 16)] & 1) << 6
                ld_row = cb0 + iota

                @plsc.parallel_loop(0, D, unroll=16)
                def _(m):
                    q = ((iota + m) & 15) + (m & 48)
                    v = plsc.load_gather(rows, [ld_row, hvec + q])
                    plsc.store_scatter(oslab, [q >> 3, q & 7, ld_row], v)

                return carry

            lax.fori_loop(0, BT // 16, per_cb, 0)

        prep_unit(0, 0)
        fire_gather(0)

        def p2_pair(i, carry):
            for p in range(2):
                u = i * 2 + p
                wait_gather(p)

                @pl.when(u + 1 < NU)
                def _():
                    prep_unit(u + 1, 1 - p)
                    fire_gather(1 - p)

                @pl.when(u >= 2)
                def _():
                    wait_out(p)

                xpose_unit(u, p)
                fire_out(u, p)

            return carry

        lax.fori_loop(0, NU // 2, p2_pair, 0)
        wait_out(0)
        wait_out(1)

    return emb(ids_t, table_t, tail_t)


def kernel(token_ids, embedding_weights):
    ids_t = token_ids.T
    table_t = embedding_weights.T
    tail_t = jnp.pad(table_t[:, TAIL0:], ((0, 0), (0, 128 - NTAIL)))
    out5 = _embedding_lookup(ids_t, table_t, tail_t)
    out = out5.transpose(0, 1, 3, 2, 4).reshape(NT, D, NB)
    return jnp.transpose(out, (2, 0, 1))


# speedup vs baseline: 1.2307x; 1.2307x over previous
import functools

import jax
import jax.numpy as jnp
from jax import lax
from jax.experimental import pallas as pl
from jax.experimental.pallas import tpu as pltpu
from jax.experimental.pallas import tpu_sc as plsc

D = 64
NE = 1000000
NPAIR = NE // 2
W1 = 256
NBLK_MAIN = NE // W1
TAIL0 = NBLK_MAIN * W1
NTAIL = NE - TAIL0
NB = 16384
NT = 20
BT = 128
NU = NT * 4


@jax.jit
def _embedding_lookup(ids_t, table_t, tail_t):
    info = plsc.get_sparse_core_info()
    num_cores, num_subcores = info.num_cores, info.num_subcores
    nw = num_cores * num_subcores
    n_k = NBLK_MAIN // nw + 2
    mesh = plsc.VectorSubcoreMesh(core_axis_name="c", subcore_axis_name="s")

    @functools.partial(
        pl.kernel,
        mesh=mesh,
        out_type=jax.ShapeDtypeStruct((NT, 8, NB // BT, 8, BT), jnp.float32),
        scratch_types=[
            pltpu.HBM((NPAIR, 2 * D), jnp.float32),
            pltpu.VMEM((D, W1), jnp.float32),
            pltpu.VMEM((D, W1), jnp.float32),
            pltpu.VMEM((W1 // 2, 2 * D), jnp.float32),
            pltpu.VMEM((W1 // 2, 2 * D), jnp.float32),
            pltpu.VMEM((NT, 4 * BT), jnp.int32),
            pltpu.VMEM((BT,), jnp.int32),
            pltpu.VMEM((BT,), jnp.int32),
            pltpu.VMEM((BT, 2 * D), jnp.float32),
            pltpu.VMEM((BT, 2 * D), jnp.float32),
            pltpu.VMEM((8, 8, BT), jnp.float32),
            pltpu.VMEM((8, 8, BT), jnp.float32),
            pltpu.SemaphoreType.DMA,
            pltpu.SemaphoreType.DMA,
            pltpu.SemaphoreType.DMA,
            pltpu.SemaphoreType.DMA,
            pltpu.SemaphoreType.DMA,
            pltpu.SemaphoreType.DMA,
            pltpu.SemaphoreType.DMA,
            pltpu.SemaphoreType.DMA,
            pltpu.SemaphoreType.REGULAR,
        ],
        compiler_params=pltpu.CompilerParams(
            use_tc_tiling_on_sc=True, needs_layout_passes=False),
    )
    def emb(ids_hbm, table_hbm, tail_hbm, out_hbm, r_hbm,
            slab0, slab1, rowbuf0, rowbuf1, idxall, idxp0, idxp1,
            rows0, rows1, oslab0, oslab1,
            sr0, sr1, sw0, sw1, g0, g1, o0, o1, csem):
        cid = lax.axis_index("c")
        sid = lax.axis_index("s")
        wid = sid * num_cores + cid

        slabs = (slab0, slab1)
        rowbufs = (rowbuf0, rowbuf1)
        srs = (sr0, sr1)
        sws = (sw0, sw1)
        idxps = (idxp0, idxp1)
        rowss = (rows0, rows1)
        oslabs = (oslab0, oslab1)
        gs = (g0, g1)
        os_ = (o0, o1)

        iota = lax.iota(jnp.int32, 16)
        half = iota >> 1
        parity64 = (iota & 1) << 6
        def rot(d):
            return (iota + d) & 15

        def blk_of(k):
            return wid + k * nw

        def fire_read(k, p):
            i0 = pl.multiple_of(blk_of(k) * W1, W1)
            pltpu.async_copy(table_hbm.at[:, pl.ds(i0, W1)], slabs[p], srs[p])

        def wait_read(p):
            pltpu.make_async_copy(
                table_hbm.at[:, pl.ds(0, W1)], slabs[p], srs[p]).wait()

        def fire_write(k, p):
            r0 = pl.multiple_of(blk_of(k) * (W1 // 2), W1 // 2)
            pltpu.async_copy(
                rowbufs[p], r_hbm.at[pl.ds(r0, W1 // 2), :], sws[p])

        def wait_write(p):
            pltpu.make_async_copy(
                rowbufs[p], r_hbm.at[pl.ds(0, W1 // 2), :], sws[p]).wait()

        def xpose_block(p):
            slab, rowbuf = slabs[p], rowbufs[p]

            def per_i0(i16, carry):
                i0 = i16 * 16
                ld_col = i0 + iota
                st_row = (i16 * 8) + half

                @plsc.parallel_loop(0, D, unroll=4)
                def _(m):
                    q = ((iota + m) & 15) + (m & 48)
                    v = plsc.load_gather(slab, [q, ld_col])
                    plsc.store_scatter(rowbuf, [st_row, parity64 + q], v)

                return carry

            lax.fori_loop(0, W1 // 16, per_i0, 0)

        @pl.when(blk_of(0) < NBLK_MAIN)
        def _():
            fire_read(0, 0)

        def p1_pair(i, carry):
            for p in range(2):
                k = i * 2 + p
                blk = blk_of(k)

                @pl.when(blk_of(k + 1) < NBLK_MAIN)
                def _():
                    fire_read(k + 1, 1 - p)

                @pl.when(blk < NBLK_MAIN)
                def _():
                    wait_read(p)

                    @pl.when(k >= 2)
                    def _():
                        wait_write(p)

                    xpose_block(p)
                    fire_write(k, p)

            return carry

        lax.fori_loop(0, n_k // 2, p1_pair, 0)
        wait_write(0)
        wait_write(1)

        @pl.when(wid == 0)
        def _tail():
            pltpu.sync_copy(tail_hbm, slab0.at[:, pl.ds(0, BT)])

            def tail_j(j, carry):
                colbase = parity64 + j
                for c0 in range(0, NTAIL, 16):
                    v = slab0[j, pl.ds(c0, 16)]
                    plsc.store_scatter(
                        rowbuf0, [(c0 >> 1) + half, colbase], v)
                return carry

            lax.fori_loop(0, D, tail_j, 0)
            pltpu.sync_copy(
                rowbuf0.at[pl.ds(0, NTAIL // 2), :],
                r_hbm.at[pl.ds(TAIL0 // 2, NTAIL // 2), :])

        plsc.subcore_barrier()
        pltpu.core_barrier(csem, core_axis_name="c")

        pltpu.sync_copy(ids_hbm.at[:, pl.ds(wid * (4 * BT), 4 * BT)], idxall)

        def prep_unit(u, p):
            t = u >> 2
            cbase = (u & 3) * BT

            def shift_blk(c, carry):
                idxps[p][pl.ds(c * 16, 16)] = (
                    idxall[t, pl.ds(cbase + c * 16, 16)] >> 1)
                return carry

            lax.fori_loop(0, BT // 16, shift_blk, 0)

        def fire_gather(p):
            pltpu.async_copy(r_hbm.at[idxps[p]], rowss[p], gs[p])

        def wait_gather(p):
            pltpu.make_async_copy(
                r_hbm.at[pl.ds(0, BT), :], rowss[p], gs[p]).wait()

        def fire_out(u, p):
            t = u >> 2
            tc = wid * 4 + (u & 3)
            pltpu.async_copy(oslabs[p], out_hbm.at[t, :, tc], os_[p])

        def wait_out(p):
            pltpu.make_async_copy(
                oslabs[p], out_hbm.at[0, :, 0], os_[p]).wait()

        def xpose_unit(u, p):
            t = u >> 2
            cbase = (u & 3) * BT
            rows, oslab = rowss[p], oslabs[p]

            def per_cb(ci, carry):
                cb0 = ci * 16
                hvec = (idxall[t, pl.ds(cbase + cb0, 16)] & 1) << 6
                ld_row = cb0 + iota

                @plsc.parallel_loop(0, D, unroll=4)
                def _(m):
                    q = ((iota + m) & 15) + (m & 48)
                    v = plsc.load_gather(rows, [ld_row, hvec + q])
                    plsc.store_scatter(oslab, [q >> 3, q & 7, ld_row], v)

                return carry

            lax.fori_loop(0, BT // 16, per_cb, 0)

        prep_unit(0, 0)
        fire_gather(0)

        def p2_pair(i, carry):
            for p in range(2):
                u = i * 2 + p
                wait_gather(p)

                @pl.when(u + 1 < NU)
                def _():
                    prep_unit(u + 1, 1 - p)
                    fire_gather(1 - p)

                @pl.when(u >= 2)
                def _():
                    wait_out(p)

                xpose_unit(u, p)
                fire_out(u, p)

            return carry

        lax.fori_loop(0, NU // 2, p2_pair, 0)
        wait_out(0)
        wait_out(1)

    return emb(ids_t, table_t, tail_t)


def kernel(token_ids, embedding_weights):
    ids_t = token_ids.T
    table_t = embedding_weights.T
    tail_t = jnp.pad(table_t[:, TAIL0:], ((0, 0), (0, 128 - NTAIL)))
    out5 = _embedding_lookup(ids_t, table_t, tail_t)
    out = out5.transpose(0, 1, 3, 2, 4).reshape(NT, D, NB)
    return jnp.transpose(out, (2, 0, 1))


# speedup vs baseline: 1.2830x; 1.0425x over previous
import functools

import jax
import jax.numpy as jnp
from jax import lax
from jax.experimental import pallas as pl
from jax.experimental.pallas import tpu as pltpu
from jax.experimental.pallas import tpu_sc as plsc

D = 64
NE = 1000000
NPAIR = NE // 2
W1 = 256
NBLK_MAIN = NE // W1
TAIL0 = NBLK_MAIN * W1
NTAIL = NE - TAIL0
NB = 16384
NT = 20
BT = 128
NU = NT * 4


@jax.jit
def _embedding_lookup(ids_t, table_t, tail_t):
    info = plsc.get_sparse_core_info()
    num_cores, num_subcores = info.num_cores, info.num_subcores
    nw = num_cores * num_subcores
    n_k = NBLK_MAIN // nw + 2
    mesh = plsc.VectorSubcoreMesh(core_axis_name="c", subcore_axis_name="s")

    @functools.partial(
        pl.kernel,
        mesh=mesh,
        out_type=jax.ShapeDtypeStruct((NT, 8, NB // BT, 8, BT), jnp.float32),
        scratch_types=[
            pltpu.HBM((NPAIR, 2 * D), jnp.float32),
            pltpu.VMEM((D, W1), jnp.float32),
            pltpu.VMEM((D, W1), jnp.float32),
            pltpu.VMEM((W1 // 2, 2 * D), jnp.float32),
            pltpu.VMEM((W1 // 2, 2 * D), jnp.float32),
            pltpu.VMEM((NT, 4 * BT), jnp.int32),
            pltpu.VMEM((BT,), jnp.int32),
            pltpu.VMEM((BT,), jnp.int32),
            pltpu.VMEM((BT, 2 * D), jnp.float32),
            pltpu.VMEM((BT, 2 * D), jnp.float32),
            pltpu.VMEM((8, 8, BT), jnp.float32),
            pltpu.VMEM((8, 8, BT), jnp.float32),
            pltpu.SemaphoreType.DMA,
            pltpu.SemaphoreType.DMA,
            pltpu.SemaphoreType.DMA,
            pltpu.SemaphoreType.DMA,
            pltpu.SemaphoreType.DMA,
            pltpu.SemaphoreType.DMA,
            pltpu.SemaphoreType.DMA,
            pltpu.SemaphoreType.DMA,
            pltpu.SemaphoreType.REGULAR,
        ],
        compiler_params=pltpu.CompilerParams(
            use_tc_tiling_on_sc=True, needs_layout_passes=False),
    )
    def emb(ids_hbm, table_hbm, tail_hbm, out_hbm, r_hbm,
            slab0, slab1, rowbuf0, rowbuf1, idxall, idxp0, idxp1,
            rows0, rows1, oslab0, oslab1,
            sr0, sr1, sw0, sw1, g0, g1, o0, o1, csem):
        cid = lax.axis_index("c")
        sid = lax.axis_index("s")
        wid = sid * num_cores + cid

        slabs = (slab0, slab1)
        rowbufs = (rowbuf0, rowbuf1)
        srs = (sr0, sr1)
        sws = (sw0, sw1)
        idxps = (idxp0, idxp1)
        rowss = (rows0, rows1)
        oslabs = (oslab0, oslab1)
        gs = (g0, g1)
        os_ = (o0, o1)

        iota = lax.iota(jnp.int32, 16)
        half = iota >> 1
        parity64 = (iota & 1) << 6
        def rot(d):
            return (iota + d) & 15

        def blk_of(k):
            return wid + k * nw

        def fire_read(k, p):
            i0 = pl.multiple_of(blk_of(k) * W1, W1)
            pltpu.async_copy(table_hbm.at[:, pl.ds(i0, W1)], slabs[p], srs[p])

        def wait_read(p):
            pltpu.make_async_copy(
                table_hbm.at[:, pl.ds(0, W1)], slabs[p], srs[p]).wait()

        def fire_write(k, p):
            r0 = pl.multiple_of(blk_of(k) * (W1 // 2), W1 // 2)
            pltpu.async_copy(
                rowbufs[p], r_hbm.at[pl.ds(r0, W1 // 2), :], sws[p])

        def wait_write(p):
            pltpu.make_async_copy(
                rowbufs[p], r_hbm.at[pl.ds(0, W1 // 2), :], sws[p]).wait()

        def xpose_block(p):
            slab, rowbuf = slabs[p], rowbufs[p]

            def per_i0(i16, carry):
                i0 = i16 * 16
                ld_col = i0 + iota
                st_row = (i16 * 8) + half

                @plsc.parallel_loop(0, D, unroll=8)
                def _(m):
                    q = ((iota + m) & 15) + (m & 48)
                    v = plsc.load_gather(slab, [q, ld_col])
                    plsc.store_scatter(rowbuf, [st_row, parity64 + q], v)

                return carry

            lax.fori_loop(0, W1 // 16, per_i0, 0)

        @pl.when(blk_of(0) < NBLK_MAIN)
        def _():
            fire_read(0, 0)

        def p1_pair(i, carry):
            for p in range(2):
                k = i * 2 + p
                blk = blk_of(k)

                @pl.when(blk_of(k + 1) < NBLK_MAIN)
                def _():
                    fire_read(k + 1, 1 - p)

                @pl.when(blk < NBLK_MAIN)
                def _():
                    wait_read(p)

                    @pl.when(k >= 2)
                    def _():
                        wait_write(p)

                    xpose_block(p)
                    fire_write(k, p)

            return carry

        lax.fori_loop(0, n_k // 2, p1_pair, 0)
        wait_write(0)
        wait_write(1)

        @pl.when(wid == 0)
        def _tail():
            pltpu.sync_copy(tail_hbm, slab0.at[:, pl.ds(0, BT)])

            def tail_j(j, carry):
                colbase = parity64 + j
                for c0 in range(0, NTAIL, 16):
                    v = slab0[j, pl.ds(c0, 16)]
                    plsc.store_scatter(
                        rowbuf0, [(c0 >> 1) + half, colbase], v)
                return carry

            lax.fori_loop(0, D, tail_j, 0)
            pltpu.sync_copy(
                rowbuf0.at[pl.ds(0, NTAIL // 2), :],
                r_hbm.at[pl.ds(TAIL0 // 2, NTAIL // 2), :])

        plsc.subcore_barrier()
        pltpu.core_barrier(csem, core_axis_name="c")

        pltpu.sync_copy(ids_hbm.at[:, pl.ds(wid * (4 * BT), 4 * BT)], idxall)

        def prep_unit(u, p):
            t = u >> 2
            cbase = (u & 3) * BT

            def shift_blk(c, carry):
                idxps[p][pl.ds(c * 16, 16)] = (
                    idxall[t, pl.ds(cbase + c * 16, 16)] >> 1)
                return carry

            lax.fori_loop(0, BT // 16, shift_blk, 0)

        def fire_gather(p):
            pltpu.async_copy(r_hbm.at[idxps[p]], rowss[p], gs[p])

        def wait_gather(p):
            pltpu.make_async_copy(
                r_hbm.at[pl.ds(0, BT), :], rowss[p], gs[p]).wait()

        def fire_out(u, p):
            t = u >> 2
            tc = wid * 4 + (u & 3)
            pltpu.async_copy(oslabs[p], out_hbm.at[t, :, tc], os_[p])

        def wait_out(p):
            pltpu.make_async_copy(
                oslabs[p], out_hbm.at[0, :, 0], os_[p]).wait()

        def xpose_unit(u, p):
            t = u >> 2
            cbase = (u & 3) * BT
            rows, oslab = rowss[p], oslabs[p]

            def per_cb(ci, carry):
                cb0 = ci * 16
                hvec = (idxall[t, pl.ds(cbase + cb0, 16)] & 1) << 6
                ld_row = cb0 + iota

                @plsc.parallel_loop(0, D, unroll=8)
                def _(m):
                    q = ((iota + m) & 15) + (m & 48)
                    v = plsc.load_gather(rows, [ld_row, hvec + q])
                    plsc.store_scatter(oslab, [q >> 3, q & 7, ld_row], v)

                return carry

            lax.fori_loop(0, BT // 16, per_cb, 0)

        prep_unit(0, 0)
        fire_gather(0)

        def p2_pair(i, carry):
            for p in range(2):
                u = i * 2 + p
                wait_gather(p)

                @pl.when(u + 1 < NU)
                def _():
                    prep_unit(u + 1, 1 - p)
                    fire_gather(1 - p)

                @pl.when(u >= 2)
                def _():
                    wait_out(p)

                xpose_unit(u, p)
                fire_out(u, p)

            return carry

        lax.fori_loop(0, NU // 2, p2_pair, 0)
        wait_out(0)
        wait_out(1)

    return emb(ids_t, table_t, tail_t)


def kernel(token_ids, embedding_weights):
    ids_t = token_ids.T
    table_t = embedding_weights.T
    tail_t = jnp.pad(table_t[:, TAIL0:], ((0, 0), (0, 128 - NTAIL)))
    out5 = _embedding_lookup(ids_t, table_t, tail_t)
    out = out5.transpose(0, 1, 3, 2, 4).reshape(NT, D, NB)
    return jnp.transpose(out, (2, 0, 1))
